# baseline (speedup 1.0000x reference)
import functools

import jax
import jax.numpy as jnp
from jax import lax
from jax.experimental import pallas as pl
from jax.experimental.pallas import tpu as pltpu
from jax.experimental.pallas import tpu_sc as plsc

E = 256
HEADS = 8
POINTS = 4
HD = 32
H = 100
W = 100
NQ = 10000
BS = 2
LANES = 128
NP = HEADS * POINTS

NWORK = 32
RPW = 312
BLK = 104
NBLK = RPW // BLK
CHUNK = 8
NCH = BLK // CHUNK
TAIL0 = NWORK * RPW


def _k1_body(v_ref, w_ref, bv_ref, o_ref, scr, scr_v, *, bb):
    h = pl.program_id(0)

    @pl.when(h == 0)
    def _stage():
        scr_v[...] = v_ref[:, bb, :]

    a = (jnp.dot(scr_v[...], w_ref[0],
                 preferred_element_type=jnp.float32) + bv_ref[0])
    scr[pl.ds(0, NQ), :] = a
    o_ref[0] = jnp.concatenate(
        [scr[pl.ds(k, NQ), :] for k in (0, 1, W, W + 1)], axis=1)


def _k2_body(q_ref, rp_ref, wxyl_ref, bxyl_ref, wx32_ref, bx32_ref,
             wy32_ref, by32_ref, gg_ref, expm_ref, idx_ref, wgt_ref, *, bb):
    qb = q_ref[:, bb, :]
    rpx = rp_ref[0][:, 0:1] * float(W) - 0.5
    rpy = rp_ref[0][:, 1:2] * float(H) - 0.5
    XYL = jnp.dot(qb, wxyl_ref[...], preferred_element_type=jnp.float32) \
        + bxyl_ref[...]
    X = XYL[:, 0:LANES] + rpx
    Y = XYL[:, LANES:2 * LANES] + rpy
    Eo = jnp.exp(XYL[:, 2 * LANES:3 * LANES])
    Sden = jnp.dot(Eo, gg_ref[...], preferred_element_type=jnp.float32)
    AW = Eo / Sden
    X32 = jnp.dot(qb, wx32_ref[...], preferred_element_type=jnp.float32) \
        + bx32_ref[...] + rpx
    Y32 = jnp.dot(qb, wy32_ref[...], preferred_element_type=jnp.float32) \
        + by32_ref[...] + rpy
    xs32 = jnp.clip(jnp.floor(X32), 0.0, float(W - 2))
    ys32 = jnp.clip(jnp.floor(Y32), 0.0, float(H - 2))
    l32 = lax.broadcasted_iota(jnp.int32, X32.shape, 1)
    plane = l32 // POINTS
    idx_ref[:, 0:NP] = (plane * (H * W) + ys32.astype(jnp.int32) * W
                        + xs32.astype(jnp.int32))
    xs128 = jnp.dot(xs32, expm_ref[...], preferred_element_type=jnp.float32)
    ys128 = jnp.dot(ys32, expm_ref[...], preferred_element_type=jnp.float32)
    l = lax.broadcasted_iota(jnp.int32, X.shape, 1)
    kx = (l % 2).astype(jnp.float32)
    ky = ((l % 4) // 2).astype(jnp.float32)
    tentx = jnp.maximum(0.0, 1.0 - jnp.abs(X - (xs128 + kx)))
    tenty = jnp.maximum(0.0, 1.0 - jnp.abs(Y - (ys128 + ky)))
    wgt_ref[...] = AW * tentx * tenty


def _k4_body(a_ref, w_ref, b_ref, q_ref, o_ref, *, bb):
    o_ref[...] = (
        jnp.dot(a_ref[...], w_ref[...], preferred_element_type=jnp.float32)
        + b_ref[...] + q_ref[:, bb, :])


def _accum_row(buf, r, wgt_v, rows_v, out_v):
    def hbody(h, carry):
        w16 = wgt_v[r, pl.ds(h * 16, 16)]
        a0 = jnp.zeros((16,), jnp.float32)
        a1 = jnp.zeros((16,), jnp.float32)
        for p in range(POINTS):
            j = h * POINTS + p
            for c in range(4):
                wsc = w16[p * 4 + c]
                a0 = a0 + wsc * rows_v[buf, j, pl.ds(c * 32, 16)]
                a1 = a1 + wsc * rows_v[buf, j, pl.ds(c * 32 + 16, 16)]
        out_v[r, pl.ds(h * 32, 16)] = a0
        out_v[r, pl.ds(h * 32 + 16, 16)] = a1
        return carry
    lax.fori_loop(0, HEADS, hbody, 0)


def _sc_body(vtp, idxp, wgtp, out, idx_v, wgt_v, rows_v, out_v, sem):
    wid = lax.axis_index("s") * 2 + lax.axis_index("c")
    base = wid * RPW

    def fire(ck, half):
        r0 = ck * CHUNK
        for c in range(CHUNK):
            pltpu.async_copy(
                vtp.at[idx_v.at[r0 + c, pl.ds(0, NP)]],
                rows_v.at[half * CHUNK + c], sem)

    def drain(ck, half):
        r0 = ck * CHUNK
        for c in range(CHUNK):
            pltpu.make_async_copy(
                vtp.at[idx_v.at[r0 + c, pl.ds(0, NP)]],
                rows_v.at[half * CHUNK + c], sem).wait()
            _accum_row(half * CHUNK + c, r0 + c, wgt_v, rows_v, out_v)

    def blk_body(blk, carry):
        b0 = base + blk * BLK
        pltpu.sync_copy(idxp.at[pl.ds(b0, BLK)], idx_v)
        pltpu.sync_copy(wgtp.at[pl.ds(b0, BLK)], wgt_v)
        fire(0, 0)

        def pair_body(jp, carry2):
            fire(2 * jp + 1, 1)
            drain(2 * jp, 0)
            fire(2 * jp + 2, 0)
            drain(2 * jp + 1, 1)
            return carry2

        lax.fori_loop(0, (NCH - 1) // 2, pair_body, 0)
        drain(NCH - 1, 0)
        pltpu.sync_copy(out_v, out.at[pl.ds(b0, BLK)])
        return carry

    lax.fori_loop(0, NBLK, blk_body, 0)

    @pl.when(wid < NQ - TAIL0)
    def _tail():
        rt = TAIL0 + wid
        pltpu.sync_copy(idxp.at[pl.ds(rt, 1)], idx_v.at[pl.ds(0, 1)])
        pltpu.sync_copy(wgtp.at[pl.ds(rt, 1)], wgt_v.at[pl.ds(0, 1)])
        pltpu.async_copy(vtp.at[idx_v.at[0, pl.ds(0, NP)]],
                         rows_v.at[0], sem).wait()
        _accum_row(0, 0, wgt_v, rows_v, out_v)
        pltpu.sync_copy(out_v.at[pl.ds(0, 1)], out.at[pl.ds(rt, 1)])


def kernel(query, value, reference_points, spatial_shapes, W_value, b_value,
           W_off, b_off, W_attn, b_attn, W_out, b_out):
    f32 = jnp.float32

    Wv3 = W_value.reshape(E, HEADS, HD).transpose(1, 0, 2)
    bv3 = b_value.reshape(HEADS, 1, HD)
    Wo3 = W_off.reshape(E, NP, 2)
    Wx32 = Wo3[:, :, 0]
    Wy32 = Wo3[:, :, 1]
    bo2 = b_off.reshape(NP, 2)
    bx32 = bo2[:, 0].reshape(1, NP)
    by32 = bo2[:, 1].reshape(1, NP)
    Wx = jnp.repeat(Wx32, 4, axis=1)
    Wy = jnp.repeat(Wy32, 4, axis=1)
    bx = jnp.repeat(bx32[0], 4).reshape(1, LANES)
    by = jnp.repeat(by32[0], 4).reshape(1, LANES)
    Wl = jnp.repeat(W_attn, 4, axis=1)
    bl = jnp.repeat(b_attn, 4).reshape(1, LANES)
    WXYL = jnp.concatenate([Wx, Wy, Wl], axis=1)
    bXYL = jnp.concatenate([bx, by, bl], axis=1)
    gidx = jnp.arange(LANES) // 16
    GG = 0.25 * (gidx[:, None] == gidx[None, :]).astype(f32)
    EXPM = (jnp.arange(NP)[:, None] == (jnp.arange(LANES)[None, :] // 4)
            ).astype(f32)
    rp3 = reference_points.reshape(BS, NQ, 2)

    mesh = plsc.VectorSubcoreMesh(core_axis_name="c", subcore_axis_name="s",
                                  num_cores=2, num_subcores=16)
    QB = 1000
    MB = 1000

    outs = []
    for bb in range(BS):
        vtp = pl.pallas_call(
            functools.partial(_k1_body, bb=bb),
            grid=(HEADS,),
            in_specs=[
                pl.BlockSpec((NQ, BS, E), lambda h: (0, 0, 0)),
                pl.BlockSpec((1, E, HD), lambda h: (h, 0, 0)),
                pl.BlockSpec((1, 1, HD), lambda h: (h, 0, 0)),
            ],
            out_specs=pl.BlockSpec((1, NQ, LANES), lambda h: (h, 0, 0)),
            out_shape=jax.ShapeDtypeStruct((HEADS, NQ, LANES), f32),
            scratch_shapes=[pltpu.VMEM((NQ + 104, HD), f32),
                            pltpu.VMEM((NQ, E), f32)],
            compiler_params=pltpu.CompilerParams(
                vmem_limit_bytes=100 * 1024 * 1024),
        )(value, Wv3, bv3)
        vtp_flat = vtp.reshape(HEADS * NQ, LANES)

        idxp, wgtp = pl.pallas_call(
            functools.partial(_k2_body, bb=bb),
            grid=(NQ // QB,),
            in_specs=[
                pl.BlockSpec((QB, BS, E), lambda i: (i, 0, 0)),
                pl.BlockSpec((1, QB, 2), lambda i, _b=bb: (_b, i, 0)),
                pl.BlockSpec((E, 3 * LANES), lambda i: (0, 0)),
                pl.BlockSpec((1, 3 * LANES), lambda i: (0, 0)),
                pl.BlockSpec((E, NP), lambda i: (0, 0)),
                pl.BlockSpec((1, NP), lambda i: (0, 0)),
                pl.BlockSpec((E, NP), lambda i: (0, 0)),
                pl.BlockSpec((1, NP), lambda i: (0, 0)),
                pl.BlockSpec((LANES, LANES), lambda i: (0, 0)),
                pl.BlockSpec((NP, LANES), lambda i: (0, 0)),
            ],
            out_specs=[
                pl.BlockSpec((QB, LANES), lambda i: (i, 0)),
                pl.BlockSpec((QB, LANES), lambda i: (i, 0)),
            ],
            out_shape=[
                jax.ShapeDtypeStruct((NQ, LANES), jnp.int32),
                jax.ShapeDtypeStruct((NQ, LANES), f32),
            ],
        )(query, rp3, WXYL, bXYL, Wx32, bx32, Wy32, by32, GG, EXPM)

        attn = pl.kernel(
            _sc_body,
            mesh=mesh,
            out_type=jax.ShapeDtypeStruct((NQ, E), f32),
            scratch_types=[
                pltpu.VMEM((BLK, LANES), jnp.int32),
                pltpu.VMEM((BLK, LANES), f32),
                pltpu.VMEM((2 * CHUNK, NP, LANES), f32),
                pltpu.VMEM((BLK, E), f32),
                pltpu.SemaphoreType.DMA,
            ],
            compiler_params=pltpu.CompilerParams(use_tc_tiling_on_sc=False),
        )(vtp_flat, idxp, wgtp)

        outb = pl.pallas_call(
            functools.partial(_k4_body, bb=bb),
            grid=(NQ // MB,),
            in_specs=[
                pl.BlockSpec((MB, E), lambda i: (i, 0)),
                pl.BlockSpec((E, E), lambda i: (0, 0)),
                pl.BlockSpec((1, E), lambda i: (0, 0)),
                pl.BlockSpec((MB, BS, E), lambda i: (i, 0, 0)),
            ],
            out_specs=pl.BlockSpec((MB, E), lambda i: (i, 0)),
            out_shape=jax.ShapeDtypeStruct((NQ, E), f32),
        )(attn, W_out, b_out.reshape(1, E), query)
        outs.append(outb)

    return jnp.stack(outs, axis=1)
